# baseline (speedup 1.0000x reference)
import jax
import jax.numpy as jnp
from jax import lax
from jax.experimental import pallas as pl
from jax.experimental.pallas import tpu as pltpu
from jax.experimental.pallas import tpu_sc as plsc

DIM = 32
N_FIELDS = 26
BATCH = 16384
NUM_CORES = 2
NUM_SUBCORES = 16
NUM_WORKERS = NUM_CORES * NUM_SUBCORES
BLK = 128
TC_PER_F = BATCH // BLK
TC_PER_W = TC_PER_F // NUM_WORKERS
BLOCKS_PER_W = N_FIELDS * TC_PER_W
P_ROWS = N_FIELDS * (DIM // 8) * TC_PER_F * 8
ROWS_PER_F = (DIM // 8) * TC_PER_F * 8

_mesh = plsc.VectorSubcoreMesh(core_axis_name="c", subcore_axis_name="s")


def _body(table_hbm, idx_hbm, p_hbm, idx_v, st_a, st_b, rows_a, rows_b, t_a, t_b, sem_g, sem_w):
    wid = lax.axis_index("s") * NUM_CORES + lax.axis_index("c")

    pltpu.sync_copy(idx_hbm.at[pl.ds(wid * (TC_PER_W * BLK), TC_PER_W * BLK), :], idx_v)

    lane = lax.iota(jnp.int32, 16)
    rowsel = [lane + 16 * k for k in range(8)]

    def stage_idx(z, st_v):
        f2 = z // TC_PER_W
        t = z % TC_PER_W
        col = jnp.full((16,), 0, jnp.int32) + f2

        @plsc.parallel_loop(0, 8, 1, unroll=8)
        def _(k):
            row = lane + (t * BLK + k * 16)
            st_v[pl.ds(k * 16, 16)] = plsc.load_gather(idx_v, [row, col])

    def gather_desc(z, st_v, rows_v):
        return pltpu.make_async_copy(table_hbm.at[st_v], rows_v, sem_g)

    def write_descs(z, t_v):
        f2 = z // TC_PER_W
        tc = wid * TC_PER_W + z % TC_PER_W
        row0 = f2 * ROWS_PER_F + tc * 8
        return [
            pltpu.make_async_copy(
                t_v.at[pl.ds(tr * 8, 8), :],
                p_hbm.at[pl.ds(row0 + tr * TC_PER_F * 8, 8), :],
                sem_w,
            )
            for tr in range(4)
        ]

    def transpose_block(rows_v, t_v):
        @plsc.parallel_loop(0, DIM, 1, unroll=4)
        def _(d):
            col = jnp.full((16,), 0, jnp.int32) + d
            for k in range(8):
                src = plsc.load_gather(rows_v, [rowsel[k], col])
                t_v[d, pl.ds(16 * k, 16)] = src

    stage_idx(0, st_a)
    gather_desc(0, st_a, rows_a).start()

    def loop_body(zz, carry):
        for sub, st_v, rows_v, t_v in (
            (0, st_a, rows_a, t_a),
            (1, st_b, rows_b, t_b),
        ):
            z = 2 * zz + sub
            gather_desc(z, st_v, rows_v).wait()

            @pl.when(z + 1 < BLOCKS_PER_W)
            def _():
                st_n = st_b if sub == 0 else st_a
                rows_n = rows_b if sub == 0 else rows_a
                stage_idx(z + 1, st_n)
                gather_desc(z + 1, st_n, rows_n).start()

            @pl.when(z >= 2)
            def _():
                for d in write_descs(z - 2, t_v):
                    d.wait()

            transpose_block(rows_v, t_v)
            for d in write_descs(z, t_v):
                d.start()
        return carry

    lax.fori_loop(0, BLOCKS_PER_W // 2, loop_body, 0)
    for d in write_descs(BLOCKS_PER_W - 2, t_a):
        d.wait()
    for d in write_descs(BLOCKS_PER_W - 1, t_b):
        d.wait()


@jax.jit
def _embed(idx_t, emb):
    k = pl.kernel(
        _body,
        mesh=_mesh,
        out_type=jax.ShapeDtypeStruct((P_ROWS, BLK), jnp.float32),
        scratch_types=[
            pltpu.VMEM((TC_PER_W * BLK, N_FIELDS), jnp.int32),
            pltpu.VMEM((BLK,), jnp.int32),
            pltpu.VMEM((BLK,), jnp.int32),
            pltpu.VMEM((BLK, DIM), jnp.float32),
            pltpu.VMEM((BLK, DIM), jnp.float32),
            pltpu.VMEM((DIM, BLK), jnp.float32),
            pltpu.VMEM((DIM, BLK), jnp.float32),
            pltpu.SemaphoreType.DMA,
            pltpu.SemaphoreType.DMA,
        ],
        compiler_params=pltpu.CompilerParams(
            use_tc_tiling_on_sc=False, needs_layout_passes=False
        ),
    )
    return k(emb, idx_t)


def kernel(token_ids, emb):
    p = _embed(token_ids, emb)
    p5 = p.reshape(N_FIELDS, DIM // 8, TC_PER_F, 8, BLK)
    return p5.transpose(2, 4, 0, 1, 3).reshape(BATCH, N_FIELDS, DIM)


# speedup vs baseline: 1.0213x; 1.0213x over previous
import jax
import jax.numpy as jnp
from jax import lax
from jax.experimental import pallas as pl
from jax.experimental.pallas import tpu as pltpu
from jax.experimental.pallas import tpu_sc as plsc

DIM = 32
N_FIELDS = 26
BATCH = 16384
NUM_CORES = 2
NUM_SUBCORES = 16
NUM_WORKERS = NUM_CORES * NUM_SUBCORES
BLK = 128
TC_PER_F = BATCH // BLK
TC_PER_W = TC_PER_F // NUM_WORKERS
BLOCKS_PER_W = N_FIELDS * TC_PER_W
P_ROWS = N_FIELDS * (DIM // 8) * TC_PER_F * 8
ROWS_PER_F = (DIM // 8) * TC_PER_F * 8

_mesh = plsc.VectorSubcoreMesh(core_axis_name="c", subcore_axis_name="s")


def _body(table_hbm, idx_hbm, p_hbm, idx_v, rows_a, rows_b, t_a, t_b, sem_g, sem_w):
    wid = lax.axis_index("s") * NUM_CORES + lax.axis_index("c")

    pltpu.sync_copy(idx_hbm.at[:, pl.ds(wid * TC_PER_W, TC_PER_W), :, :], idx_v)

    lane = lax.iota(jnp.int32, 16)
    rowsel = [lane + 16 * k for k in range(8)]

    def gather_desc(z, rows_v):
        f2 = z // TC_PER_W
        t = z % TC_PER_W
        return pltpu.make_async_copy(
            table_hbm.at[idx_v.at[f2 // 8, t, f2 % 8]], rows_v, sem_g
        )

    def write_descs(z, t_v):
        f2 = z // TC_PER_W
        tc = wid * TC_PER_W + z % TC_PER_W
        row0 = f2 * ROWS_PER_F + tc * 8
        return [
            pltpu.make_async_copy(
                t_v.at[pl.ds(tr * 8, 8), :],
                p_hbm.at[pl.ds(row0 + tr * TC_PER_F * 8, 8), :],
                sem_w,
            )
            for tr in range(4)
        ]

    def transpose_block(rows_v, t_v):
        @plsc.parallel_loop(0, DIM, 1, unroll=4)
        def _(d):
            col = jnp.full((16,), 0, jnp.int32) + d
            for k in range(8):
                src = plsc.load_gather(rows_v, [rowsel[k], col])
                t_v[d, pl.ds(16 * k, 16)] = src

    gather_desc(0, rows_a).start()

    def loop_body(zz, carry):
        for sub, rows_v, t_v in ((0, rows_a, t_a), (1, rows_b, t_b)):
            z = 2 * zz + sub
            gather_desc(z, rows_v).wait()

            @pl.when(z + 1 < BLOCKS_PER_W)
            def _():
                gather_desc(z + 1, rows_b if sub == 0 else rows_a).start()

            @pl.when(z >= 2)
            def _():
                for d in write_descs(z - 2, t_v):
                    d.wait()

            transpose_block(rows_v, t_v)
            for d in write_descs(z, t_v):
                d.start()
        return carry

    lax.fori_loop(0, BLOCKS_PER_W // 2, loop_body, 0)
    for d in write_descs(BLOCKS_PER_W - 2, t_a):
        d.wait()
    for d in write_descs(BLOCKS_PER_W - 1, t_b):
        d.wait()


@jax.jit
def _embed(idx_t, emb):
    k = pl.kernel(
        _body,
        mesh=_mesh,
        out_type=jax.ShapeDtypeStruct((P_ROWS, BLK), jnp.float32),
        scratch_types=[
            pltpu.VMEM((4, TC_PER_W, 8, BLK), jnp.int32),
            pltpu.VMEM((BLK, DIM), jnp.float32),
            pltpu.VMEM((BLK, DIM), jnp.float32),
            pltpu.VMEM((DIM, BLK), jnp.float32),
            pltpu.VMEM((DIM, BLK), jnp.float32),
            pltpu.SemaphoreType.DMA,
            pltpu.SemaphoreType.DMA,
        ],
        compiler_params=pltpu.CompilerParams(
            use_tc_tiling_on_sc=False, needs_layout_passes=False
        ),
    )
    return k(emb, idx_t)


def kernel(token_ids, emb):
    idx4 = (
        jnp.pad(token_ids.T, ((0, 8 * (N_FIELDS // 8 + 1) - N_FIELDS), (0, 0)))
        .reshape(4, 8, TC_PER_F, BLK)
        .transpose(0, 2, 1, 3)
    )
    p = _embed(idx4, emb)
    p5 = p.reshape(N_FIELDS, DIM // 8, TC_PER_F, 8, BLK)
    return p5.transpose(2, 4, 0, 1, 3).reshape(BATCH, N_FIELDS, DIM)
